# baseline (speedup 1.0000x reference)
import functools

import jax
import jax.numpy as jnp
from jax import lax
from jax.experimental import pallas as pl
from jax.experimental.pallas import tpu as pltpu
from jax.experimental.pallas import tpu_sc as plsc

EMB = 16
CHUNK = 128


def _shuffle(x, perm):
    return lax.gather(
        x,
        perm[:, None],
        lax.GatherDimensionNumbers(
            offset_dims=(),
            collapsed_slice_dims=(0,),
            start_index_map=(0,),
        ),
        slice_sizes=(1,),
        mode=lax.GatherScatterMode.PROMISE_IN_BOUNDS,
    )


def _make_sc_kernel(B):
    info = plsc.get_sparse_core_info()
    NC, NS = info.num_cores, info.num_subcores
    NW = NC * NS
    b_per_w = B // NW
    n_chunks = b_per_w // CHUNK
    mesh = plsc.VectorSubcoreMesh(core_axis_name="c", subcore_axis_name="s")

    @functools.partial(
        pl.kernel,
        mesh=mesh,
        compiler_params=pltpu.CompilerParams(use_tc_tiling_on_sc=False),
        out_type=jax.ShapeDtypeStruct((B,), jnp.float32),
        scratch_types=[
            pltpu.VMEM((n_chunks, CHUNK), jnp.int32),
            pltpu.VMEM((n_chunks, CHUNK), jnp.int32),
            pltpu.VMEM((n_chunks, CHUNK, EMB), jnp.float32),
            pltpu.VMEM((n_chunks, CHUNK, EMB), jnp.float32),
            pltpu.VMEM((b_per_w,), jnp.float32),
            pltpu.SemaphoreType.DMA,
        ],
    )
    def k(u_hbm, i_hbm, uf_hbm, if_hbm, out_hbm,
          uidx_v, iidx_v, urows_v, irows_v, out_v, sem):
        wid = lax.axis_index("s") * NC + lax.axis_index("c")
        base = wid * b_per_w

        for j in range(n_chunks):
            pltpu.sync_copy(u_hbm.at[pl.ds(base + j * CHUNK, CHUNK)],
                            uidx_v.at[j])
            pltpu.sync_copy(i_hbm.at[pl.ds(base + j * CHUNK, CHUNK)],
                            iidx_v.at[j])

        copies = []
        for j in range(n_chunks):
            copies.append(pltpu.async_copy(uf_hbm.at[uidx_v.at[j]],
                                           urows_v.at[j], sem))
            copies.append(pltpu.async_copy(if_hbm.at[iidx_v.at[j]],
                                           irows_v.at[j], sem))
        for c in copies:
            c.wait()

        lane = lax.iota(jnp.int32, 16)
        folds = [jnp.bitwise_xor(lane, k) for k in (8, 4, 2, 1)]
        for j in range(n_chunks):
            def body(g, _, j=j):
                row0 = g * 16
                acc = jnp.zeros((16,), jnp.float32)
                for r in range(16):
                    u = urows_v[j, row0 + r, :]
                    it = irows_v[j, row0 + r, :]
                    s = u * it
                    for perm in folds:
                        s = s + _shuffle(s, perm)
                    acc = jnp.where(lane == r, s, acc)
                out_v[pl.ds(j * CHUNK + row0, 16)] = acc
                return 0
            lax.fori_loop(0, CHUNK // 16, body, 0)

        pltpu.sync_copy(out_v, out_hbm.at[pl.ds(base, b_per_w)])

    return k


def kernel(u_id, i_id, user_factors, item_factors):
    B = u_id.shape[0]
    k = _make_sc_kernel(B)
    return k(u_id, i_id, user_factors, item_factors)


# speedup vs baseline: 3.3963x; 3.3963x over previous
import functools

import jax
import jax.numpy as jnp
from jax import lax
from jax.experimental import pallas as pl
from jax.experimental.pallas import tpu as pltpu
from jax.experimental.pallas import tpu_sc as plsc

EMB = 16
WIN = 1024
PAD = 128
NW = 32


def _shuffle(x, perm):
    return lax.gather(
        x,
        perm[:, None],
        lax.GatherDimensionNumbers(
            offset_dims=(),
            collapsed_slice_dims=(0,),
            start_index_map=(0,),
        ),
        slice_sizes=(1,),
        mode=lax.GatherScatterMode.PROMISE_IN_BOUNDS,
    )


def _make_extract(B, n_users, n_items):
    u_tc8 = n_users // WIN
    u_tail0 = u_tc8 * WIN
    i_tc8 = n_items // WIN
    i_tail0 = i_tc8 * WIN
    mesh = plsc.VectorSubcoreMesh(core_axis_name="c", subcore_axis_name="s")

    @functools.partial(
        pl.kernel,
        mesh=mesh,
        compiler_params=pltpu.CompilerParams(
            use_tc_tiling_on_sc=True, needs_layout_passes=False),
        out_type=(jax.ShapeDtypeStruct(((B + NW) * EMB,), jnp.float32),
                  jax.ShapeDtypeStruct(((B + NW) * EMB,), jnp.float32)),
        scratch_types=[
            pltpu.VMEM((B,), jnp.int32),
            pltpu.VMEM((B,), jnp.int32),
            pltpu.VMEM((B,), jnp.int32),
            pltpu.VMEM((B,), jnp.int32),
            pltpu.VMEM((B,), jnp.int32),
            pltpu.VMEM((EMB, PAD + WIN), jnp.float32),
            pltpu.VMEM((64 * 16 * EMB,), jnp.float32),
            pltpu.SemaphoreType.DMA,
            pltpu.SemaphoreType.DMA,
        ],
    )
    def k(u_hbm, i_hbm, ut_hbm, it_hbm, utail_hbm, itail_hbm,
          gu_hbm, gi_hbm,
          idv, hr, hb, br, bb, wb, outbox, wsem, osem):
        wid = lax.axis_index("s") * 2 + lax.axis_index("c")
        lane = lax.iota(jnp.int32, 16)

        NB = 64

        def drain_vreg():
            pltpu.make_async_copy(
                u_hbm.at[pl.ds(0, 16 * EMB)],
                outbox.at[pl.ds(0, 16 * EMB)], osem).wait()

        def process_window(col_lo, cnt, out_hbm, gv0):
            def bpass(v, bcnt):
                r = hr[pl.ds(v * 16, 16)]
                b = hb[pl.ds(v * 16, 16)]
                m = (r >= col_lo) & (r < col_lo + WIN)
                m = m & (v * 16 + lane < cnt)
                c = plsc.all_reduce_population_count(m)[0]
                plsc.store_compressed(br.at[pl.ds(bcnt, 16)], r, mask=m)
                plsc.store_compressed(bb.at[pl.ds(bcnt, 16)], b, mask=m)
                return bcnt + c
            nhv = (cnt + 15) // 16
            bcnt = lax.fori_loop(0, nhv, bpass, 0)

            def ext(v, gv):
                @pl.when(gv >= NB - 1)
                def _():
                    drain_vreg()
                rv = br[pl.ds(v * 16, 16)]
                bv = bb[pl.ds(v * 16, 16)]
                rem = bcnt - v * 16
                bank = (gv % NB) * (16 * EMB)
                jv = jnp.clip(rv - col_lo, 0, WIN - 1)
                dstv = jnp.where(lane < rem, bv, B + wid) * EMB
                for l in range(16):
                    j = jv[l]
                    jt = j & ~15
                    jl = jnp.full((16,), j & 15, jnp.int32)
                    acc = jnp.zeros((16,), jnp.float32)
                    for c in range(EMB):
                        vc = wb[c, pl.ds(PAD + jt, 16)]
                        acc = jnp.where(lane == c, _shuffle(vc, jl), acc)
                    outbox[pl.ds(bank + l * EMB, EMB)] = acc
                for l in range(16):
                    pltpu.async_copy(
                        outbox.at[pl.ds(bank + l * EMB, EMB)],
                        out_hbm.at[pl.ds(pl.multiple_of(dstv[l], 8), EMB)],
                        osem)
                return gv + 1
            return lax.fori_loop(0, (bcnt + 15) // 16, ext, gv0)

        def phase(id_hbm, t_hbm, tail_hbm, out_hbm, n_tc8, tail0, tail_wid):
            nw_lo = n_tc8 // NW
            extra = n_tc8 % NW
            nw = jnp.where(wid < extra, nw_lo + 1, nw_lo)
            w0_idx = jnp.where(wid < extra,
                               wid * (nw_lo + 1),
                               extra * (nw_lo + 1) + (wid - extra) * nw_lo)
            lo = w0_idx * WIN
            hi = lo + nw * WIN

            pltpu.sync_copy(id_hbm, idv)

            def scan(v, cnt):
                r = idv[pl.ds(v * 16, 16)]
                bvec = v * 16 + lane
                m = (r >= lo) & (r < hi)
                m = m | ((wid == tail_wid) & (r >= tail0))
                c = plsc.all_reduce_population_count(m)[0]
                plsc.store_compressed(hr.at[pl.ds(cnt, 16)], r, mask=m)
                plsc.store_compressed(hb.at[pl.ds(cnt, 16)], bvec, mask=m)
                return cnt + c
            cnt = lax.fori_loop(0, B // 16, scan, 0)

            def win_body(widx, gv):
                col0 = pl.multiple_of((w0_idx + widx) * WIN, 128)
                pltpu.async_copy(t_hbm.at[:, pl.ds(col0, WIN)],
                                 wb.at[:, pl.ds(PAD, WIN)], wsem).wait()
                return process_window(col0, cnt, out_hbm, gv)
            gv = lax.fori_loop(0, nw, win_body, 0)

            def tail_body(_, gv_t):
                pltpu.async_copy(tail_hbm, wb.at[:, pl.ds(PAD, WIN)],
                                 wsem).wait()
                return process_window(tail0, cnt, out_hbm, gv_t)
            gv = lax.fori_loop(
                0, jnp.where(wid == tail_wid, 1, 0), tail_body, gv)

            lax.fori_loop(0, jnp.minimum(gv, NB - 1),
                          lambda _, x: (drain_vreg(), x)[1], 0)

        phase(u_hbm, ut_hbm, utail_hbm, gu_hbm, u_tc8, u_tail0, 0)
        phase(i_hbm, it_hbm, itail_hbm, gi_hbm, i_tc8, i_tail0, 1)

    return k


def _make_join(B):
    b_per_w = B // NW
    mesh = plsc.VectorSubcoreMesh(core_axis_name="c", subcore_axis_name="s")

    @functools.partial(
        pl.kernel,
        mesh=mesh,
        out_type=jax.ShapeDtypeStruct((B,), jnp.float32),
        scratch_types=[
            pltpu.VMEM((b_per_w * EMB,), jnp.float32),
            pltpu.VMEM((b_per_w * EMB,), jnp.float32),
            pltpu.VMEM((b_per_w,), jnp.float32),
        ],
    )
    def k(gu_hbm, gi_hbm, out_hbm, ubuf, ibuf, out_v):
        wid = lax.axis_index("s") * 2 + lax.axis_index("c")
        base = wid * b_per_w
        lane = lax.iota(jnp.int32, 16)
        folds = [jnp.bitwise_xor(lane, f) for f in (8, 4, 2, 1)]

        pltpu.sync_copy(gu_hbm.at[pl.ds(base * EMB, b_per_w * EMB)], ubuf)
        pltpu.sync_copy(gi_hbm.at[pl.ds(base * EMB, b_per_w * EMB)], ibuf)

        def body(g, _):
            row0 = g * 16
            acc = jnp.zeros((16,), jnp.float32)
            for r in range(16):
                u = ubuf[pl.ds((row0 + r) * EMB, EMB)]
                it = ibuf[pl.ds((row0 + r) * EMB, EMB)]
                s = u * it
                for perm in folds:
                    s = s + _shuffle(s, perm)
                acc = jnp.where(lane == r, s, acc)
            out_v[pl.ds(row0, 16)] = acc
            return 0
        lax.fori_loop(0, b_per_w // 16, body, 0)

        pltpu.sync_copy(out_v, out_hbm.at[pl.ds(base, b_per_w)])

    return k


def kernel(u_id, i_id, user_factors, item_factors):
    B = u_id.shape[0]
    n_users, n_items = user_factors.shape[0], item_factors.shape[0]
    u_tail0 = (n_users // WIN) * WIN
    i_tail0 = (n_items // WIN) * WIN
    ut_tail = jnp.pad(user_factors[u_tail0:].T,
                      ((0, 0), (0, WIN - (n_users - u_tail0))))
    it_tail = jnp.pad(item_factors[i_tail0:].T,
                      ((0, 0), (0, WIN - (n_items - i_tail0))))
    gu, gi = _make_extract(B, n_users, n_items)(
        u_id, i_id, user_factors.T, item_factors.T, ut_tail, it_tail)
    return _make_join(B)(gu, gi)
